# baseline (speedup 1.0000x reference)
import jax
import jax.numpy as jnp
from jax import lax
from jax.experimental import pallas as pl
from jax.experimental.pallas import tpu as pltpu
from jax.experimental.pallas import tpu_sc as plsc

L = 2048
H = 16
TT = 4096
NC = 2
NS = 16
BLK = 16
NBUF = 2
VL = 16
ROWS_PER_W = (H * L) // (NC * NS)


def _sc_body(tt_hbm, out_hbm, tt_v, stage, sem_in, sem_out):
    cid = lax.axis_index("c")
    sid = lax.axis_index("s")
    wid = sid * NC + cid
    h = wid // 2
    half = wid % 2
    base_i = half * ROWS_PER_W

    pltpu.async_copy(tt_hbm.at[pl.ds(h * TT, TT)], tt_v, sem_in).wait()

    def assemble(buf, i0):
        o0 = (L - 1) - i0

        def col_body(c, carry):
            s = c * VL
            for rr in range(BLK):
                stage[buf, rr, pl.ds(s, VL)] = tt_v[pl.ds(o0 - rr + s, VL)]
            return carry

        lax.fori_loop(0, L // VL, col_body, 0)

    def drain_one():
        pltpu.make_async_copy(
            stage.at[0],
            out_hbm.at[0, 0, pl.ds(0, BLK)],
            sem_out,
        ).wait()

    n_pairs = ROWS_PER_W // BLK // NBUF

    def loop_body(g, carry):
        for buf in range(NBUF):
            i0 = base_i + (g * NBUF + buf) * BLK

            @pl.when(g > 0)
            def _():
                drain_one()

            assemble(buf, i0)
            pltpu.async_copy(
                stage.at[buf],
                out_hbm.at[0, h, pl.ds(pl.multiple_of(i0, BLK), BLK)],
                sem_out,
            )
        return carry

    lax.fori_loop(0, n_pairs, loop_body, 0)
    for _ in range(NBUF):
        drain_one()


@jax.jit
def _run_sc(tt):
    mesh = plsc.VectorSubcoreMesh(
        core_axis_name="c", subcore_axis_name="s", num_cores=NC, num_subcores=NS
    )
    return pl.kernel(
        _sc_body,
        out_type=jax.ShapeDtypeStruct((1, H, L, L), jnp.float32),
        mesh=mesh,
        scratch_types=[
            pltpu.VMEM((TT,), jnp.float32),
            pltpu.VMEM((NBUF, BLK, L), jnp.float32),
            pltpu.SemaphoreType.DMA,
            pltpu.SemaphoreType.DMA,
        ],
    )(tt)


def kernel(x, bias_table):
    del x
    tt = jnp.transpose(jnp.pad(bias_table, ((0, TT - (2 * L - 1)), (0, 0))))
    return _run_sc(tt.reshape(H * TT))


# speedup vs baseline: 4.5540x; 4.5540x over previous
import jax
import jax.numpy as jnp
from jax import lax
from jax.experimental import pallas as pl
from jax.experimental.pallas import tpu as pltpu
from jax.experimental.pallas import tpu_sc as plsc

L = 2048
H = 16
TT = 4096
NC = 2
NS = 16
BLK = 16
NBUF = 2
VL = 16
ROWS_PER_W = (H * L) // (NC * NS)


def _sc_body(tt_hbm, out_hbm, tt_v, stage, sem_in, sem_out):
    cid = lax.axis_index("c")
    sid = lax.axis_index("s")
    wid = sid * NC + cid
    h = wid // 2
    half = wid % 2
    base_i = half * ROWS_PER_W

    pltpu.async_copy(tt_hbm.at[pl.ds(h * TT, TT)], tt_v, sem_in).wait()

    def assemble(buf, i0):
        o0 = (L - 1) - i0

        @plsc.parallel_loop(0, L // VL, unroll=2)
        def _(c):
            s = c * VL
            for rr in range(BLK):
                stage[buf, rr, pl.ds(s, VL)] = tt_v[pl.ds(o0 - rr + s, VL)]

    def drain_one():
        pltpu.make_async_copy(
            stage.at[0],
            out_hbm.at[0, 0, pl.ds(0, BLK)],
            sem_out,
        ).wait()

    n_pairs = ROWS_PER_W // BLK // NBUF

    def loop_body(g, carry):
        for buf in range(NBUF):
            i0 = base_i + (g * NBUF + buf) * BLK

            @pl.when(g > 0)
            def _():
                drain_one()

            assemble(buf, i0)
            pltpu.async_copy(
                stage.at[buf],
                out_hbm.at[0, h, pl.ds(pl.multiple_of(i0, BLK), BLK)],
                sem_out,
            )
        return carry

    lax.fori_loop(0, n_pairs, loop_body, 0)
    for _ in range(NBUF):
        drain_one()


@jax.jit
def _run_sc(tt):
    mesh = plsc.VectorSubcoreMesh(
        core_axis_name="c", subcore_axis_name="s", num_cores=NC, num_subcores=NS
    )
    return pl.kernel(
        _sc_body,
        out_type=jax.ShapeDtypeStruct((1, H, L, L), jnp.float32),
        mesh=mesh,
        scratch_types=[
            pltpu.VMEM((TT,), jnp.float32),
            pltpu.VMEM((NBUF, BLK, L), jnp.float32),
            pltpu.SemaphoreType.DMA,
            pltpu.SemaphoreType.DMA,
        ],
    )(tt)


def kernel(x, bias_table):
    del x
    tt = jnp.transpose(jnp.pad(bias_table, ((0, TT - (2 * L - 1)), (0, 0))))
    return _run_sc(tt.reshape(H * TT))


# speedup vs baseline: 4.5680x; 1.0031x over previous
import jax
import jax.numpy as jnp
from jax import lax
from jax.experimental import pallas as pl
from jax.experimental.pallas import tpu as pltpu
from jax.experimental.pallas import tpu_sc as plsc

L = 2048
H = 16
TT = 4096
NC = 2
NS = 16
BLK = 16
NBUF = 2
VL = 16
ROWS_PER_W = (H * L) // (NC * NS)


def _sc_body(tt_hbm, out_hbm, tt_v, stage, sem_in, sem_out):
    cid = lax.axis_index("c")
    sid = lax.axis_index("s")
    wid = sid * NC + cid
    h = wid // 2
    half = wid % 2
    base_i = half * ROWS_PER_W

    pltpu.async_copy(tt_hbm.at[pl.ds(h * TT, TT)], tt_v, sem_in).wait()

    def assemble(buf, i0):
        o0 = (L - 1) - i0

        @plsc.parallel_loop(0, L // VL, unroll=4)
        def _(c):
            s = c * VL
            for rr in range(BLK):
                stage[buf, rr, pl.ds(s, VL)] = tt_v[pl.ds(o0 - rr + s, VL)]

    def drain_one():
        pltpu.make_async_copy(
            stage.at[0],
            out_hbm.at[0, 0, pl.ds(0, BLK)],
            sem_out,
        ).wait()

    n_pairs = ROWS_PER_W // BLK // NBUF

    def loop_body(g, carry):
        for buf in range(NBUF):
            i0 = base_i + (g * NBUF + buf) * BLK

            @pl.when(g > 0)
            def _():
                drain_one()

            assemble(buf, i0)
            pltpu.async_copy(
                stage.at[buf],
                out_hbm.at[0, h, pl.ds(pl.multiple_of(i0, BLK), BLK)],
                sem_out,
            )
        return carry

    lax.fori_loop(0, n_pairs, loop_body, 0)
    for _ in range(NBUF):
        drain_one()


@jax.jit
def _run_sc(tt):
    mesh = plsc.VectorSubcoreMesh(
        core_axis_name="c", subcore_axis_name="s", num_cores=NC, num_subcores=NS
    )
    return pl.kernel(
        _sc_body,
        out_type=jax.ShapeDtypeStruct((1, H, L, L), jnp.float32),
        mesh=mesh,
        scratch_types=[
            pltpu.VMEM((TT,), jnp.float32),
            pltpu.VMEM((NBUF, BLK, L), jnp.float32),
            pltpu.SemaphoreType.DMA,
            pltpu.SemaphoreType.DMA,
        ],
    )(tt)


def kernel(x, bias_table):
    del x
    tt = jnp.transpose(jnp.pad(bias_table, ((0, TT - (2 * L - 1)), (0, 0))))
    return _run_sc(tt.reshape(H * TT))
